# baseline (device time: 27097 ns/iter reference)
import jax
import jax.numpy as jnp
from jax import lax
from jax.experimental import pallas as pl
from jax.experimental.pallas import tpu as pltpu

N_DEV = 16


def kernel(x, Wq, K_ext, V_ext, Wo):
    B, Sq, D = x.shape
    _, Skv, Hq, Dh = K_ext.shape
    Dm = Hq * Dh
    Do = Wo.shape[1]
    QB = Sq // 64
    ROWS = Sq // N_DEV
    OPB = 64 // ROWS

    Kf = K_ext.reshape(B, Skv, Dm)
    Vf = V_ext.reshape(B, Skv, Dm)

    def body(x_ref, wq_ref, k_ref, v_ref, wo_ref, out_ref,
             accC, accE, recvC, recvE, accRC, accRE, ybuf,
             s1C, s1E, r1C, r1E, s2, r2):
        me = lax.axis_index("i")

        barrier = pltpu.get_barrier_semaphore()
        for j in range(N_DEV):
            @pl.when(j != me)
            def _(j=j):
                pl.semaphore_signal(
                    barrier, inc=1, device_id=(j,),
                    device_id_type=pl.DeviceIdType.MESH)
        pl.semaphore_wait(barrier, N_DEV - 1)

        Qms = [jnp.dot(x_ref[b], wq_ref[...],
                       preferred_element_type=jnp.float32) for b in range(B)]

        p1_started = []
        for q in range(QB):
            rows = pl.ds(q * 64, 64)
            for b in range(B):
                ses = []
                for h in range(Hq):
                    cols = pl.ds(h * Dh, Dh)
                    Qh = Qms[b][q * 64:(q + 1) * 64, h * Dh:(h + 1) * Dh]
                    Kh = k_ref[b, rows, cols]
                    Vh = v_ref[b, rows, cols]
                    s = lax.dot_general(
                        Qh, Kh, (((1,), (1,)), ((), ())),
                        preferred_element_type=jnp.float32) * 0.125
                    w = jnp.exp(s)
                    ses.append(jnp.sum(w, axis=1))
                    ctx = lax.dot_general(
                        w, Vh, (((1,), (0,)), ((), ())),
                        preferred_element_type=jnp.float32)
                    accC[b, rows, cols] = ctx.astype(jnp.bfloat16)
                accE[b, rows, :] = jnp.stack(ses, axis=1).astype(jnp.bfloat16)

            for t in range(OPB):
                owner = q * OPB + t
                osl = pl.ds(owner * ROWS, ROWS)

                @pl.when(owner != me)
                def _(owner=owner, osl=osl):
                    rC = pltpu.make_async_remote_copy(
                        src_ref=accC.at[:, osl, :],
                        dst_ref=recvC.at[me],
                        send_sem=s1C.at[owner], recv_sem=r1C.at[me],
                        device_id=(owner,),
                        device_id_type=pl.DeviceIdType.MESH)
                    rE = pltpu.make_async_remote_copy(
                        src_ref=accE.at[:, osl, :],
                        dst_ref=recvE.at[me],
                        send_sem=s1E.at[owner], recv_sem=r1E.at[me],
                        device_id=(owner,),
                        device_id_type=pl.DeviceIdType.MESH)
                    rC.start()
                    rE.start()

                @pl.when(owner == me)
                def _(osl=osl):
                    accRC[...] = accC[:, osl, :].astype(jnp.float32)
                    accRE[...] = accE[:, osl, :].astype(jnp.float32)

        for j in range(N_DEV):
            @pl.when(j != me)
            def _(j=j):
                pltpu.make_async_remote_copy(
                    src_ref=accC.at[:, pl.ds(0, ROWS), :],
                    dst_ref=recvC.at[j],
                    send_sem=s1C.at[j], recv_sem=r1C.at[j],
                    device_id=(j,), device_id_type=pl.DeviceIdType.MESH,
                ).wait_recv()
                pltpu.make_async_remote_copy(
                    src_ref=accE.at[:, pl.ds(0, ROWS), :],
                    dst_ref=recvE.at[j],
                    send_sem=s1E.at[j], recv_sem=r1E.at[j],
                    device_id=(j,), device_id_type=pl.DeviceIdType.MESH,
                ).wait_recv()
                accRC[...] += recvC[j].astype(jnp.float32)
                accRE[...] += recvE[j].astype(jnp.float32)

        Nrm = accRC[...].reshape(B, ROWS, Hq, Dh) / accRE[...][..., None]
        ybuf[:, pl.ds(me * ROWS, ROWS), :] = (
            Nrm.reshape(B, ROWS, Dm).astype(jnp.bfloat16))

        for j in range(N_DEV):
            @pl.when(j != me)
            def _(j=j):
                pltpu.make_async_remote_copy(
                    src_ref=ybuf.at[:, pl.ds(me * ROWS, ROWS), :],
                    dst_ref=ybuf.at[:, pl.ds(me * ROWS, ROWS), :],
                    send_sem=s2.at[j], recv_sem=r2.at[me],
                    device_id=(j,),
                    device_id_type=pl.DeviceIdType.MESH).start()

        for b in range(B):
            out_ref[b, pl.ds(me * ROWS, ROWS), :] = jnp.dot(
                ybuf[b, pl.ds(me * ROWS, ROWS), :].astype(jnp.float32),
                wo_ref[...], preferred_element_type=jnp.float32)

        for j in range(N_DEV):
            jsl = pl.ds(j * ROWS, ROWS)

            @pl.when(j != me)
            def _(j=j, jsl=jsl):
                pltpu.make_async_remote_copy(
                    src_ref=ybuf.at[:, jsl, :],
                    dst_ref=ybuf.at[:, jsl, :],
                    send_sem=s2.at[j], recv_sem=r2.at[j],
                    device_id=(j,), device_id_type=pl.DeviceIdType.MESH,
                ).wait_recv()
                for b in range(B):
                    out_ref[b, jsl, :] = jnp.dot(
                        ybuf[b, jsl, :].astype(jnp.float32), wo_ref[...],
                        preferred_element_type=jnp.float32)

        for j in range(N_DEV):
            @pl.when(j != me)
            def _(j=j):
                pltpu.make_async_remote_copy(
                    src_ref=accC.at[:, pl.ds(0, ROWS), :],
                    dst_ref=recvC.at[j],
                    send_sem=s1C.at[j], recv_sem=r1C.at[j],
                    device_id=(j,), device_id_type=pl.DeviceIdType.MESH,
                ).wait_send()
                pltpu.make_async_remote_copy(
                    src_ref=accE.at[:, pl.ds(0, ROWS), :],
                    dst_ref=recvE.at[j],
                    send_sem=s1E.at[j], recv_sem=r1E.at[j],
                    device_id=(j,), device_id_type=pl.DeviceIdType.MESH,
                ).wait_send()
                pltpu.make_async_remote_copy(
                    src_ref=ybuf.at[:, pl.ds(0, ROWS), :],
                    dst_ref=ybuf.at[:, pl.ds(0, ROWS), :],
                    send_sem=s2.at[j], recv_sem=r2.at[j],
                    device_id=(j,), device_id_type=pl.DeviceIdType.MESH,
                ).wait_send()

    out_shape = jax.ShapeDtypeStruct((B, Sq, Do), jnp.float32)
    return pl.pallas_call(
        body,
        out_shape=out_shape,
        in_specs=[pl.BlockSpec(memory_space=pltpu.VMEM)] * 5,
        out_specs=pl.BlockSpec(memory_space=pltpu.VMEM),
        scratch_shapes=[
            pltpu.VMEM((B, Sq, Dm), jnp.bfloat16),
            pltpu.VMEM((B, Sq, Hq), jnp.bfloat16),
            pltpu.VMEM((N_DEV, B, ROWS, Dm), jnp.bfloat16),
            pltpu.VMEM((N_DEV, B, ROWS, Hq), jnp.bfloat16),
            pltpu.VMEM((B, ROWS, Dm), jnp.float32),
            pltpu.VMEM((B, ROWS, Hq), jnp.float32),
            pltpu.VMEM((B, Sq, Dm), jnp.bfloat16),
            pltpu.SemaphoreType.DMA((N_DEV,)),
            pltpu.SemaphoreType.DMA((N_DEV,)),
            pltpu.SemaphoreType.DMA((N_DEV,)),
            pltpu.SemaphoreType.DMA((N_DEV,)),
            pltpu.SemaphoreType.DMA((N_DEV,)),
            pltpu.SemaphoreType.DMA((N_DEV,)),
        ],
        compiler_params=pltpu.CompilerParams(collective_id=0),
    )(x, Wq, Kf, Vf, Wo)


# device time: 21702 ns/iter; 1.2486x vs baseline; 1.2486x over previous
import jax
import jax.numpy as jnp
from jax import lax
from jax.experimental import pallas as pl
from jax.experimental.pallas import tpu as pltpu

N_DEV = 16


def kernel(x, Wq, K_ext, V_ext, Wo):
    B, Sq, D = x.shape
    _, Skv, Hq, Dh = K_ext.shape
    Dm = Hq * Dh
    Do = Wo.shape[1]
    QB = Sq // 64
    ROWS = Sq // N_DEV
    OPB = 64 // ROWS

    Kf = K_ext.reshape(B, Skv, Dm)
    Vf = V_ext.reshape(B, Skv, Dm)

    def body(x_ref, wq_ref, k_ref, v_ref, wo_ref, out_ref,
             accC, accE, recvC, recvE, accRC, accRE, ybuf,
             s1C, s1E, r1C, r1E, s2, r2):
        me = lax.axis_index("i")

        barrier = pltpu.get_barrier_semaphore()
        for j in range(N_DEV):
            @pl.when(j != me)
            def _(j=j):
                pl.semaphore_signal(
                    barrier, inc=1, device_id=(j,),
                    device_id_type=pl.DeviceIdType.MESH)
        pl.semaphore_wait(barrier, N_DEV - 1)

        bf16 = jnp.bfloat16
        wq16 = wq_ref[...].astype(bf16)
        qb = lax.broadcasted_iota(jnp.int32, (Sq, Skv), 0) // 64
        kb = lax.broadcasted_iota(jnp.int32, (Sq, Skv), 1) // 64
        maskf = jnp.where(qb == kb, 1.0, 0.0).astype(jnp.float32)
        Qms = [lax.dot_general(x_ref[b].astype(bf16), wq16,
                               (((1,), (0,)), ((), ())),
                               preferred_element_type=jnp.float32
                               ).astype(bf16) for b in range(B)]

        for b in range(B):
            ses = []
            for h in range(Hq):
                cols = pl.ds(h * Dh, Dh)
                Kh = k_ref[b, :, cols].astype(bf16)
                Vh = v_ref[b, :, cols].astype(bf16)
                S = lax.dot_general(
                    Qms[b][:, h * Dh:(h + 1) * Dh], Kh,
                    (((1,), (1,)), ((), ())),
                    preferred_element_type=jnp.float32)
                w = jnp.exp(S * 0.125) * maskf
                ses.append(jnp.sum(w, axis=1))
                ctx = lax.dot_general(
                    w.astype(bf16), Vh, (((1,), (0,)), ((), ())),
                    preferred_element_type=jnp.float32)
                accC[b, :, cols] = ctx.astype(bf16)
            accE[b, :, :] = jnp.stack(ses, axis=1).astype(bf16)

        for q in range(QB):
            for t in range(OPB):
                owner = q * OPB + t
                osl = pl.ds(owner * ROWS, ROWS)

                @pl.when(owner != me)
                def _(owner=owner, osl=osl):
                    rC = pltpu.make_async_remote_copy(
                        src_ref=accC.at[:, osl, :],
                        dst_ref=recvC.at[me],
                        send_sem=s1C.at[owner], recv_sem=r1C.at[me],
                        device_id=(owner,),
                        device_id_type=pl.DeviceIdType.MESH)
                    rE = pltpu.make_async_remote_copy(
                        src_ref=accE.at[:, osl, :],
                        dst_ref=recvE.at[me],
                        send_sem=s1E.at[owner], recv_sem=r1E.at[me],
                        device_id=(owner,),
                        device_id_type=pl.DeviceIdType.MESH)
                    rC.start()
                    rE.start()

                @pl.when(owner == me)
                def _(osl=osl):
                    accRC[...] = accC[:, osl, :].astype(jnp.float32)
                    accRE[...] = accE[:, osl, :].astype(jnp.float32)

        for j in range(N_DEV):
            @pl.when(j != me)
            def _(j=j):
                pltpu.make_async_remote_copy(
                    src_ref=accC.at[:, pl.ds(0, ROWS), :],
                    dst_ref=recvC.at[j],
                    send_sem=s1C.at[j], recv_sem=r1C.at[j],
                    device_id=(j,), device_id_type=pl.DeviceIdType.MESH,
                ).wait_recv()
                pltpu.make_async_remote_copy(
                    src_ref=accE.at[:, pl.ds(0, ROWS), :],
                    dst_ref=recvE.at[j],
                    send_sem=s1E.at[j], recv_sem=r1E.at[j],
                    device_id=(j,), device_id_type=pl.DeviceIdType.MESH,
                ).wait_recv()
                accRC[...] += recvC[j].astype(jnp.float32)
                accRE[...] += recvE[j].astype(jnp.float32)

        Nrm = accRC[...].reshape(B, ROWS, Hq, Dh) / accRE[...][..., None]
        ybuf[:, pl.ds(me * ROWS, ROWS), :] = (
            Nrm.reshape(B, ROWS, Dm).astype(jnp.bfloat16))

        for j in range(N_DEV):
            @pl.when(j != me)
            def _(j=j):
                pltpu.make_async_remote_copy(
                    src_ref=ybuf.at[:, pl.ds(me * ROWS, ROWS), :],
                    dst_ref=ybuf.at[:, pl.ds(me * ROWS, ROWS), :],
                    send_sem=s2.at[j], recv_sem=r2.at[me],
                    device_id=(j,),
                    device_id_type=pl.DeviceIdType.MESH).start()

        wo16 = wo_ref[...].astype(bf16)
        for b in range(B):
            out_ref[b, pl.ds(me * ROWS, ROWS), :] = lax.dot_general(
                ybuf[b, pl.ds(me * ROWS, ROWS), :], wo16,
                (((1,), (0,)), ((), ())),
                preferred_element_type=jnp.float32)

        for j in range(N_DEV):
            jsl = pl.ds(j * ROWS, ROWS)

            @pl.when(j != me)
            def _(j=j, jsl=jsl):
                pltpu.make_async_remote_copy(
                    src_ref=ybuf.at[:, jsl, :],
                    dst_ref=ybuf.at[:, jsl, :],
                    send_sem=s2.at[j], recv_sem=r2.at[j],
                    device_id=(j,), device_id_type=pl.DeviceIdType.MESH,
                ).wait_recv()
                for b in range(B):
                    out_ref[b, jsl, :] = lax.dot_general(
                        ybuf[b, jsl, :], wo16, (((1,), (0,)), ((), ())),
                        preferred_element_type=jnp.float32)

        for j in range(N_DEV):
            @pl.when(j != me)
            def _(j=j):
                pltpu.make_async_remote_copy(
                    src_ref=accC.at[:, pl.ds(0, ROWS), :],
                    dst_ref=recvC.at[j],
                    send_sem=s1C.at[j], recv_sem=r1C.at[j],
                    device_id=(j,), device_id_type=pl.DeviceIdType.MESH,
                ).wait_send()
                pltpu.make_async_remote_copy(
                    src_ref=accE.at[:, pl.ds(0, ROWS), :],
                    dst_ref=recvE.at[j],
                    send_sem=s1E.at[j], recv_sem=r1E.at[j],
                    device_id=(j,), device_id_type=pl.DeviceIdType.MESH,
                ).wait_send()
                pltpu.make_async_remote_copy(
                    src_ref=ybuf.at[:, pl.ds(0, ROWS), :],
                    dst_ref=ybuf.at[:, pl.ds(0, ROWS), :],
                    send_sem=s2.at[j], recv_sem=r2.at[j],
                    device_id=(j,), device_id_type=pl.DeviceIdType.MESH,
                ).wait_send()

    out_shape = jax.ShapeDtypeStruct((B, Sq, Do), jnp.float32)
    return pl.pallas_call(
        body,
        out_shape=out_shape,
        in_specs=[pl.BlockSpec(memory_space=pltpu.VMEM)] * 5,
        out_specs=pl.BlockSpec(memory_space=pltpu.VMEM),
        scratch_shapes=[
            pltpu.VMEM((B, Sq, Dm), jnp.bfloat16),
            pltpu.VMEM((B, Sq, Hq), jnp.bfloat16),
            pltpu.VMEM((N_DEV, B, ROWS, Dm), jnp.bfloat16),
            pltpu.VMEM((N_DEV, B, ROWS, Hq), jnp.bfloat16),
            pltpu.VMEM((B, ROWS, Dm), jnp.float32),
            pltpu.VMEM((B, ROWS, Hq), jnp.float32),
            pltpu.VMEM((B, Sq, Dm), jnp.bfloat16),
            pltpu.SemaphoreType.DMA((N_DEV,)),
            pltpu.SemaphoreType.DMA((N_DEV,)),
            pltpu.SemaphoreType.DMA((N_DEV,)),
            pltpu.SemaphoreType.DMA((N_DEV,)),
            pltpu.SemaphoreType.DMA((N_DEV,)),
            pltpu.SemaphoreType.DMA((N_DEV,)),
        ],
        compiler_params=pltpu.CompilerParams(collective_id=0),
    )(x, Wq, Kf, Vf, Wo)
